# baseline (speedup 1.0000x reference)
import functools

import jax
import jax.numpy as jnp
from jax import lax
from jax.experimental import pallas as pl
from jax.experimental.pallas import tpu as pltpu
from jax.experimental.pallas import tpu_sc as plsc

N_NODES = 10000
N_HEDGES = 10000
N_EDGES = 320000
IVD = 128
VD = 128
QD = 64
ED = 128
NCLS = 40

_NC = 2
_NS = 16
_L = 16
_NW = _NC * _NS
_RPT = N_NODES // _NS

_CA = 512
_CHW_A = 20
_CB = 128
_CHW_B = 80
_NSUP = 4
_CSUP = _CHW_B // _NSUP
E_PAD = _NW * _CHW_A * _CA

_mesh = plsc.VectorSubcoreMesh(
    core_axis_name="c", subcore_axis_name="s", num_cores=_NC, num_subcores=_NS
)
_sc_params = pltpu.CompilerParams(needs_layout_passes=False,
                                  use_tc_tiling_on_sc=False)


def _worker_id():
    return lax.axis_index("s") * _NC + lax.axis_index("c")


@functools.partial(
    pl.kernel,
    out_type=[
        jax.ShapeDtypeStruct((E_PAD,), jnp.float32),
        jax.ShapeDtypeStruct((_NW, _L), jnp.float32),
    ],
    mesh=_mesh,
    scratch_types=[
        pltpu.VMEM((_CHW_A, 2 * _CA), jnp.int32),
        pltpu.VMEM((_CHW_A * _CA,), jnp.float32),
        pltpu.VMEM((2 * _CA, QD // 2), jnp.int32),
        pltpu.VMEM((2 * _CA, QD // 2), jnp.int32),
        pltpu.VMEM((_L,), jnp.float32),
        pltpu.SemaphoreType.DMA,
        pltpu.SemaphoreType.DMA,
    ],
    compiler_params=_sc_params,
)
def _edge_attn(kq_hbm, idx_hbm, attn_hbm, wmax_hbm,
               idxall, attnall, kqb0, kqb1, maxv, gsem0, gsem1):
    wid = _worker_id()
    crow0 = wid * _CHW_A
    eidx0 = lax.iota(jnp.int32, _L)

    pltpu.sync_copy(idx_hbm.at[pl.ds(crow0, _CHW_A)], idxall)

    def issue(ci, kqb, sem):
        pltpu.async_copy(kq_hbm.at[idxall.at[ci]], kqb, sem)

    def wait(ci, kqb, sem):
        pltpu.make_async_copy(kq_hbm.at[idxall.at[ci]], kqb, sem).wait()

    def compute(ci, kqb, macc):
        def grp(g, macc):
            e0 = g * _L
            ei = eidx0 + e0
            eiq = ei + _CA
            accs = [jnp.zeros((_L,), jnp.float32) for _ in range(4)]
            for dp in range(QD // 2):
                dv = jnp.full((_L,), dp, jnp.int32)
                kw = plsc.bitcast(plsc.load_gather(kqb, [ei, dv]), jnp.bfloat16)
                qw = plsc.bitcast(plsc.load_gather(kqb, [eiq, dv]), jnp.bfloat16)
                ka, kb2 = plsc.unpack(kw, format=plsc.PackFormat.INTERLEAVED)
                qa, qb2 = plsc.unpack(qw, format=plsc.PackFormat.INTERLEAVED)
                accs[(2 * dp) % 4] = accs[(2 * dp) % 4] + ka * qa
                accs[(2 * dp + 1) % 4] = accs[(2 * dp + 1) % 4] + kb2 * qb2
            acc = (accs[0] + accs[1]) + (accs[2] + accs[3])
            a = jnp.where(acc >= 0.0, acc, acc * 0.01) * 0.125
            attnall[pl.ds(ci * _CA + e0, _L)] = a
            return jnp.maximum(macc, a)

        return lax.fori_loop(0, _CA // _L, grp, macc)

    issue(0, kqb0, gsem0)
    issue(1, kqb1, gsem1)

    def pair(p, macc):
        c0 = 2 * p
        wait(c0, kqb0, gsem0)
        macc = compute(c0, kqb0, macc)

        @pl.when(p < _CHW_A // 2 - 1)
        def _():
            issue(c0 + 2, kqb0, gsem0)

        c1 = 2 * p + 1
        wait(c1, kqb1, gsem1)
        macc = compute(c1, kqb1, macc)

        @pl.when(p < _CHW_A // 2 - 1)
        def _():
            issue(c1 + 2, kqb1, gsem1)

        return macc

    macc = lax.fori_loop(0, _CHW_A // 2, pair,
                         jnp.full((_L,), -1e30, jnp.float32))
    pltpu.sync_copy(attnall, attn_hbm.at[pl.ds(crow0 * _CA, _CHW_A * _CA)])
    maxv[...] = macc
    pltpu.sync_copy(maxv, wmax_hbm.at[wid])


@functools.partial(
    pl.kernel,
    out_type=[
        jax.ShapeDtypeStruct((_NC, N_NODES, VD), jnp.float32),
        jax.ShapeDtypeStruct((_NC, N_NODES, 8), jnp.float32),
    ],
    mesh=_mesh,
    scratch_types=[
        pltpu.VMEM((_CSUP, _CB), jnp.int32),
        pltpu.VMEM((_CSUP, _CB), jnp.int32),
        pltpu.VMEM((_CSUP, _CB), jnp.float32),
        pltpu.VMEM((_CB, VD // 2), jnp.int32),
        pltpu.VMEM((_CB, VD // 2), jnp.int32),
        pltpu.VMEM((_CB, VD), jnp.float32),
        pltpu.VMEM((_CB,), jnp.float32),
        pltpu.VMEM((_CB, 8), jnp.float32),
        pltpu.VMEM((_NW, _L), jnp.float32),
        pltpu.VMEM_SHARED((N_NODES, VD), jnp.float32),
        pltpu.VMEM_SHARED((N_NODES, 8), jnp.float32),
        pltpu.SemaphoreType.DMA,
        pltpu.SemaphoreType.DMA,
    ],
    compiler_params=_sc_params,
)
def _edge_agg(v_hbm, src_hbm, dst_hbm, attn_hbm, wmax_hbm, acc_hbm, den_hbm,
              srcs, dsts, attns, vb0, vb1, msg, exv, exb, wmaxv, accum, denacc,
              gsem0, gsem1):
    cid = lax.axis_index("c")
    sid = lax.axis_index("s")
    wid = sid * _NC + cid
    eidx0 = lax.iota(jnp.int32, _L)
    zero16 = jnp.zeros((_L,), jnp.float32)
    zidx = jnp.zeros((_L,), jnp.int32)

    pltpu.sync_copy(wmax_hbm, wmaxv)
    m = wmaxv[0]
    for r in range(1, _NW):
        m = jnp.maximum(m, wmaxv[r])
    gmax = jnp.max(m)

    def zrow(e, carry):
        for j in range(VD // _L):
            msg[e, pl.ds(j * _L, _L)] = zero16
        return carry

    lax.fori_loop(0, _CB, zrow, 0)
    for g in range(_CB // _L):
        for d in range(8):
            plsc.store_scatter(exb, [eidx0 + g * _L, jnp.full((_L,), d, jnp.int32)],
                               zero16)
    row0 = sid * _RPT
    _zc = [(o, min(_CB, _RPT - o)) for o in range(0, _RPT, _CB)]
    for off, sz in _zc:
        pltpu.sync_copy(msg.at[pl.ds(0, sz)], accum.at[pl.ds(row0 + off, sz)])
        pltpu.sync_copy(exb.at[pl.ds(0, sz)], denacc.at[pl.ds(row0 + off, sz)])
    plsc.subcore_barrier()

    def issue(ci, vb, sem):
        pltpu.async_copy(v_hbm.at[srcs.at[ci]], vb, sem)

    def wait(ci, vb, sem):
        pltpu.make_async_copy(v_hbm.at[srcs.at[ci]], vb, sem).wait()

    for sup in range(_NSUP):
        crow = wid * _CHW_B + sup * _CSUP
        pltpu.sync_copy(src_hbm.at[pl.ds(crow, _CSUP)], srcs)
        pltpu.sync_copy(dst_hbm.at[pl.ds(crow, _CSUP)], dsts)
        pltpu.sync_copy(attn_hbm.at[pl.ds(crow, _CSUP)], attns)
        gbase = (wid * _CHW_B + sup * _CSUP) * _CB

        issue(0, vb0, gsem0)
        issue(1, vb1, gsem1)

        def process(ci, vb, sem):
            wait(ci, vb, sem)

            def grp(g, carry):
                e0 = g * _L
                ei = eidx0 + e0
                egid = (gbase + ci * _CB + e0) + eidx0
                ex = jnp.exp(attns[ci, pl.ds(e0, _L)] - gmax)
                ex = jnp.where(egid < N_EDGES, ex, 0.0)
                exv[pl.ds(e0, _L)] = ex
                plsc.store_scatter(exb, [ei, zidx], ex)
                return carry

            lax.fori_loop(0, _CB // _L, grp, 0)

            def edge(e, carry):
                exbc = plsc.load_gather(exv, [jnp.full((_L,), e, jnp.int32)])
                for t in range(VD // 32):
                    w = plsc.bitcast(vb[e, pl.ds(t * _L, _L)], jnp.bfloat16)
                    a, b = plsc.unpack(w, format=plsc.PackFormat.INTERLEAVED)
                    msg[e, pl.ds(t * 32, _L)] = a * exbc
                    msg[e, pl.ds(t * 32 + _L, _L)] = b * exbc
                return carry

            lax.fori_loop(0, _CB, edge, 0)

            @pl.when(ci < _CSUP - 2)
            def _():
                issue(ci + 2, vb, sem)

            pltpu.sync_copy(msg, accum.at[dsts.at[ci]], add=True)
            pltpu.sync_copy(exb, denacc.at[dsts.at[ci]], add=True)

        def pairb(p, carry):
            process(2 * p, vb0, gsem0)
            process(2 * p + 1, vb1, gsem1)
            return carry

        lax.fori_loop(0, _CSUP // 2, pairb, 0)

    plsc.subcore_barrier()
    for off, sz in _zc:
        pltpu.sync_copy(accum.at[pl.ds(row0 + off, sz)],
                        acc_hbm.at[cid, pl.ds(row0 + off, sz)])
        pltpu.sync_copy(denacc.at[pl.ds(row0 + off, sz)],
                        den_hbm.at[cid, pl.ds(row0 + off, sz)])


_ROWS = 1000


def _proj_body(vf, ef, wv1t, bv1, we1t, be1, wqvt, bqv, wket, bke, wvet, bve,
               fe_o, q_o, k_o, v_o):
    fv = jnp.dot(vf[...], wv1t[...], preferred_element_type=jnp.float32) + bv1[...]
    fe = jnp.dot(ef[...], we1t[...], preferred_element_type=jnp.float32) + be1[...]
    fe_o[...] = fe
    q_o[...] = jnp.dot(fv, wqvt[...], preferred_element_type=jnp.float32) + bqv[...]
    k_o[...] = jnp.dot(fe, wket[...], preferred_element_type=jnp.float32) + bke[...]
    v_o[...] = jnp.dot(fe, wvet[...], preferred_element_type=jnp.float32) + bve[...]


def _final_body(acc, den, wclst, bcls, h_o, pred_o):
    s = acc[0] + acc[1]
    denom = den[0, :, 0:1] + den[1, :, 0:1]
    h = s / (denom + 1e-9)
    h_o[...] = h
    pred_o[...] = jnp.dot(h, wclst[...], preferred_element_type=jnp.float32) + bcls[...]


def _row_spec(cols):
    return pl.BlockSpec((_ROWS, cols), lambda i: (i, 0))


def _full_spec(shape):
    nd = len(shape)
    return pl.BlockSpec(shape, lambda i: (0,) * nd)


def kernel(vfeat, efeat, edge_index, W_v1, b_v1, W_e1, b_e1, W_qv, b_qv,
           W_ke, b_ke, W_ve, b_ve, W_cls, b_cls, first_layer, last_layer):
    pad = E_PAD - N_EDGES
    src = jnp.concatenate([edge_index[0], jnp.zeros((pad,), jnp.int32)])
    dst = jnp.concatenate([edge_index[1], jnp.zeros((pad,), jnp.int32)])
    src_a = src.reshape(E_PAD // _CA, _CA)
    dst_a = dst.reshape(E_PAD // _CA, _CA)
    src_b = src.reshape(E_PAD // _CB, _CB)
    dst_b = dst.reshape(E_PAD // _CB, _CB)
    grid = N_NODES // _ROWS

    feat_e, q, k, v = pl.pallas_call(
        _proj_body,
        grid=(grid,),
        in_specs=[
            _row_spec(IVD),
            _row_spec(IED := efeat.shape[1]),
            _full_spec((IVD, VD)), _full_spec((1, VD)),
            _full_spec((IED, ED)), _full_spec((1, ED)),
            _full_spec((VD, QD)), _full_spec((1, QD)),
            _full_spec((ED, QD)), _full_spec((1, QD)),
            _full_spec((ED, VD)), _full_spec((1, VD)),
        ],
        out_specs=[_row_spec(ED), _row_spec(QD), _row_spec(QD), _row_spec(VD)],
        out_shape=[
            jax.ShapeDtypeStruct((N_HEDGES, ED), jnp.float32),
            jax.ShapeDtypeStruct((N_NODES, QD), jnp.float32),
            jax.ShapeDtypeStruct((N_HEDGES, QD), jnp.float32),
            jax.ShapeDtypeStruct((N_HEDGES, VD), jnp.float32),
        ],
    )(vfeat, efeat,
      W_v1.T, b_v1.reshape(1, VD),
      W_e1.T, b_e1.reshape(1, ED),
      W_qv.T, b_qv.reshape(1, QD),
      W_ke.T, b_ke.reshape(1, QD),
      W_ve.T, b_ve.reshape(1, VD))

    kq = jnp.concatenate([k, q], axis=0)
    kq_p = jax.lax.bitcast_convert_type(
        kq.astype(jnp.bfloat16).reshape(2 * N_HEDGES, QD // 2, 2), jnp.int32)
    v3 = v.astype(jnp.bfloat16).reshape(N_HEDGES, VD // 32, 2, _L)
    v_p = jax.lax.bitcast_convert_type(
        jnp.stack([v3[:, :, 0, :], v3[:, :, 1, :]], axis=-1),
        jnp.int32).reshape(N_HEDGES, VD // 2)
    idx_a = jnp.concatenate([src_a, dst_a + N_HEDGES], axis=1)
    attn, wmax = _edge_attn(kq_p, idx_a)
    attn_b = attn.reshape(E_PAD // _CB, _CB)
    acc, den = _edge_agg(v_p, src_b, dst_b, attn_b, wmax)

    h, pred = pl.pallas_call(
        _final_body,
        grid=(grid,),
        in_specs=[
            pl.BlockSpec((_NC, _ROWS, VD), lambda i: (0, i, 0)),
            pl.BlockSpec((_NC, _ROWS, 8), lambda i: (0, i, 0)),
            _full_spec((VD, NCLS)), _full_spec((1, NCLS)),
        ],
        out_specs=[_row_spec(VD), _row_spec(NCLS)],
        out_shape=[
            jax.ShapeDtypeStruct((N_NODES, VD), jnp.float32),
            jax.ShapeDtypeStruct((N_NODES, NCLS), jnp.float32),
        ],
    )(acc, den, W_cls.T, b_cls.reshape(1, NCLS))

    return (h, feat_e, pred)


# speedup vs baseline: 1.0108x; 1.0108x over previous
import functools

import jax
import jax.numpy as jnp
from jax import lax
from jax.experimental import pallas as pl
from jax.experimental.pallas import tpu as pltpu
from jax.experimental.pallas import tpu_sc as plsc

N_NODES = 10000
N_HEDGES = 10000
N_EDGES = 320000
IVD = 128
VD = 128
QD = 64
ED = 128
NCLS = 40

_NC = 2
_NS = 16
_L = 16
_NW = _NC * _NS
_RPT = N_NODES // _NS

_CA = 256
_CHW_A = 40
_CB = 128
_CHW_B = 80
_NSUP = 4
_CSUP = _CHW_B // _NSUP
E_PAD = _NW * _CHW_A * _CA

_mesh = plsc.VectorSubcoreMesh(
    core_axis_name="c", subcore_axis_name="s", num_cores=_NC, num_subcores=_NS
)
_sc_params = pltpu.CompilerParams(needs_layout_passes=False,
                                  use_tc_tiling_on_sc=False)


def _worker_id():
    return lax.axis_index("s") * _NC + lax.axis_index("c")


@functools.partial(
    pl.kernel,
    out_type=[
        jax.ShapeDtypeStruct((E_PAD,), jnp.float32),
        jax.ShapeDtypeStruct((_NW, _L), jnp.float32),
    ],
    mesh=_mesh,
    scratch_types=[
        pltpu.VMEM((_CHW_A, 2 * _CA), jnp.int32),
        pltpu.VMEM((_CHW_A * _CA,), jnp.float32),
        pltpu.VMEM((2 * _CA, QD // 2), jnp.int32),
        pltpu.VMEM((2 * _CA, QD // 2), jnp.int32),
        pltpu.VMEM((_L,), jnp.float32),
        pltpu.SemaphoreType.DMA,
        pltpu.SemaphoreType.DMA,
    ],
    compiler_params=_sc_params,
)
def _edge_attn(kq_hbm, idx_hbm, attn_hbm, wmax_hbm,
               idxall, attnall, kqb0, kqb1, maxv, gsem0, gsem1):
    wid = _worker_id()
    crow0 = wid * _CHW_A
    eidx0 = lax.iota(jnp.int32, _L)

    pltpu.sync_copy(idx_hbm.at[pl.ds(crow0, _CHW_A)], idxall)

    def issue(ci, kqb, sem):
        pltpu.async_copy(kq_hbm.at[idxall.at[ci]], kqb, sem)

    def wait(ci, kqb, sem):
        pltpu.make_async_copy(kq_hbm.at[idxall.at[ci]], kqb, sem).wait()

    def compute(ci, kqb, macc):
        def grp(g, macc):
            e0 = g * _L
            ei = eidx0 + e0
            eiq = ei + _CA
            accs = [jnp.zeros((_L,), jnp.float32) for _ in range(4)]
            for dp in range(QD // 2):
                dv = jnp.full((_L,), dp, jnp.int32)
                kw = plsc.bitcast(plsc.load_gather(kqb, [ei, dv]), jnp.bfloat16)
                qw = plsc.bitcast(plsc.load_gather(kqb, [eiq, dv]), jnp.bfloat16)
                ka, kb2 = plsc.unpack(kw, format=plsc.PackFormat.INTERLEAVED)
                qa, qb2 = plsc.unpack(qw, format=plsc.PackFormat.INTERLEAVED)
                accs[(2 * dp) % 4] = accs[(2 * dp) % 4] + ka * qa
                accs[(2 * dp + 1) % 4] = accs[(2 * dp + 1) % 4] + kb2 * qb2
            acc = (accs[0] + accs[1]) + (accs[2] + accs[3])
            a = jnp.where(acc >= 0.0, acc, acc * 0.01) * 0.125
            attnall[pl.ds(ci * _CA + e0, _L)] = a
            return jnp.maximum(macc, a)

        return lax.fori_loop(0, _CA // _L, grp, macc)

    issue(0, kqb0, gsem0)
    issue(1, kqb1, gsem1)

    def pair(p, macc):
        c0 = 2 * p
        wait(c0, kqb0, gsem0)
        macc = compute(c0, kqb0, macc)

        @pl.when(p < _CHW_A // 2 - 1)
        def _():
            issue(c0 + 2, kqb0, gsem0)

        c1 = 2 * p + 1
        wait(c1, kqb1, gsem1)
        macc = compute(c1, kqb1, macc)

        @pl.when(p < _CHW_A // 2 - 1)
        def _():
            issue(c1 + 2, kqb1, gsem1)

        return macc

    macc = lax.fori_loop(0, _CHW_A // 2, pair,
                         jnp.full((_L,), -1e30, jnp.float32))
    pltpu.sync_copy(attnall, attn_hbm.at[pl.ds(crow0 * _CA, _CHW_A * _CA)])
    maxv[...] = macc
    pltpu.sync_copy(maxv, wmax_hbm.at[wid])


@functools.partial(
    pl.kernel,
    out_type=[
        jax.ShapeDtypeStruct((_NC, N_NODES, VD), jnp.float32),
        jax.ShapeDtypeStruct((_NC, N_NODES, 8), jnp.float32),
    ],
    mesh=_mesh,
    scratch_types=[
        pltpu.VMEM((_CSUP, _CB), jnp.int32),
        pltpu.VMEM((_CSUP, _CB), jnp.int32),
        pltpu.VMEM((_CSUP, _CB), jnp.float32),
        pltpu.VMEM((_CB, VD // 2), jnp.int32),
        pltpu.VMEM((_CB, VD // 2), jnp.int32),
        pltpu.VMEM((_CB, VD), jnp.float32),
        pltpu.VMEM((_CB,), jnp.float32),
        pltpu.VMEM((_CB, 8), jnp.float32),
        pltpu.VMEM((_NW, _L), jnp.float32),
        pltpu.VMEM_SHARED((N_NODES, VD), jnp.float32),
        pltpu.VMEM_SHARED((N_NODES, 8), jnp.float32),
        pltpu.SemaphoreType.DMA,
        pltpu.SemaphoreType.DMA,
    ],
    compiler_params=_sc_params,
)
def _edge_agg(v_hbm, src_hbm, dst_hbm, attn_hbm, wmax_hbm, acc_hbm, den_hbm,
              srcs, dsts, attns, vb0, vb1, msg, exv, exb, wmaxv, accum, denacc,
              gsem0, gsem1):
    cid = lax.axis_index("c")
    sid = lax.axis_index("s")
    wid = sid * _NC + cid
    eidx0 = lax.iota(jnp.int32, _L)
    zero16 = jnp.zeros((_L,), jnp.float32)
    zidx = jnp.zeros((_L,), jnp.int32)

    pltpu.sync_copy(wmax_hbm, wmaxv)
    m = wmaxv[0]
    for r in range(1, _NW):
        m = jnp.maximum(m, wmaxv[r])
    gmax = jnp.max(m)

    def zrow(e, carry):
        for j in range(VD // _L):
            msg[e, pl.ds(j * _L, _L)] = zero16
        return carry

    lax.fori_loop(0, _CB, zrow, 0)
    for g in range(_CB // _L):
        for d in range(8):
            plsc.store_scatter(exb, [eidx0 + g * _L, jnp.full((_L,), d, jnp.int32)],
                               zero16)
    row0 = sid * _RPT
    _zc = [(o, min(_CB, _RPT - o)) for o in range(0, _RPT, _CB)]
    for off, sz in _zc:
        pltpu.sync_copy(msg.at[pl.ds(0, sz)], accum.at[pl.ds(row0 + off, sz)])
        pltpu.sync_copy(exb.at[pl.ds(0, sz)], denacc.at[pl.ds(row0 + off, sz)])
    plsc.subcore_barrier()

    def issue(ci, vb, sem):
        pltpu.async_copy(v_hbm.at[srcs.at[ci]], vb, sem)

    def wait(ci, vb, sem):
        pltpu.make_async_copy(v_hbm.at[srcs.at[ci]], vb, sem).wait()

    for sup in range(_NSUP):
        crow = wid * _CHW_B + sup * _CSUP
        pltpu.sync_copy(src_hbm.at[pl.ds(crow, _CSUP)], srcs)
        pltpu.sync_copy(dst_hbm.at[pl.ds(crow, _CSUP)], dsts)
        pltpu.sync_copy(attn_hbm.at[pl.ds(crow, _CSUP)], attns)
        gbase = (wid * _CHW_B + sup * _CSUP) * _CB

        issue(0, vb0, gsem0)
        issue(1, vb1, gsem1)

        def process(ci, vb, sem):
            wait(ci, vb, sem)

            def grp(g, carry):
                e0 = g * _L
                ei = eidx0 + e0
                egid = (gbase + ci * _CB + e0) + eidx0
                ex = jnp.exp(attns[ci, pl.ds(e0, _L)] - gmax)
                ex = jnp.where(egid < N_EDGES, ex, 0.0)
                exv[pl.ds(e0, _L)] = ex
                plsc.store_scatter(exb, [ei, zidx], ex)
                return carry

            lax.fori_loop(0, _CB // _L, grp, 0)

            def edge(e, carry):
                exbc = plsc.load_gather(exv, [jnp.full((_L,), e, jnp.int32)])
                for t in range(VD // 32):
                    w = plsc.bitcast(vb[e, pl.ds(t * _L, _L)], jnp.bfloat16)
                    a, b = plsc.unpack(w, format=plsc.PackFormat.INTERLEAVED)
                    msg[e, pl.ds(t * 32, _L)] = a * exbc
                    msg[e, pl.ds(t * 32 + _L, _L)] = b * exbc
                return carry

            lax.fori_loop(0, _CB, edge, 0)

            @pl.when(ci < _CSUP - 2)
            def _():
                issue(ci + 2, vb, sem)

            pltpu.sync_copy(msg, accum.at[dsts.at[ci]], add=True)
            pltpu.sync_copy(exb, denacc.at[dsts.at[ci]], add=True)

        def pairb(p, carry):
            process(2 * p, vb0, gsem0)
            process(2 * p + 1, vb1, gsem1)
            return carry

        lax.fori_loop(0, _CSUP // 2, pairb, 0)

    plsc.subcore_barrier()
    for off, sz in _zc:
        pltpu.sync_copy(accum.at[pl.ds(row0 + off, sz)],
                        acc_hbm.at[cid, pl.ds(row0 + off, sz)])
        pltpu.sync_copy(denacc.at[pl.ds(row0 + off, sz)],
                        den_hbm.at[cid, pl.ds(row0 + off, sz)])


_ROWS = 1000


def _proj_body(vf, ef, wv1t, bv1, we1t, be1, wqvt, bqv, wket, bke, wvet, bve,
               fe_o, q_o, k_o, v_o):
    fv = jnp.dot(vf[...], wv1t[...], preferred_element_type=jnp.float32) + bv1[...]
    fe = jnp.dot(ef[...], we1t[...], preferred_element_type=jnp.float32) + be1[...]
    fe_o[...] = fe
    q_o[...] = jnp.dot(fv, wqvt[...], preferred_element_type=jnp.float32) + bqv[...]
    k_o[...] = jnp.dot(fe, wket[...], preferred_element_type=jnp.float32) + bke[...]
    v_o[...] = jnp.dot(fe, wvet[...], preferred_element_type=jnp.float32) + bve[...]


def _final_body(acc, den, wclst, bcls, h_o, pred_o):
    s = acc[0] + acc[1]
    denom = den[0, :, 0:1] + den[1, :, 0:1]
    h = s / (denom + 1e-9)
    h_o[...] = h
    pred_o[...] = jnp.dot(h, wclst[...], preferred_element_type=jnp.float32) + bcls[...]


def _row_spec(cols):
    return pl.BlockSpec((_ROWS, cols), lambda i: (i, 0))


def _full_spec(shape):
    nd = len(shape)
    return pl.BlockSpec(shape, lambda i: (0,) * nd)


def kernel(vfeat, efeat, edge_index, W_v1, b_v1, W_e1, b_e1, W_qv, b_qv,
           W_ke, b_ke, W_ve, b_ve, W_cls, b_cls, first_layer, last_layer):
    pad = E_PAD - N_EDGES
    src = jnp.concatenate([edge_index[0], jnp.zeros((pad,), jnp.int32)])
    dst = jnp.concatenate([edge_index[1], jnp.zeros((pad,), jnp.int32)])
    src_a = src.reshape(E_PAD // _CA, _CA)
    dst_a = dst.reshape(E_PAD // _CA, _CA)
    src_b = src.reshape(E_PAD // _CB, _CB)
    dst_b = dst.reshape(E_PAD // _CB, _CB)
    grid = N_NODES // _ROWS

    feat_e, q, k, v = pl.pallas_call(
        _proj_body,
        grid=(grid,),
        in_specs=[
            _row_spec(IVD),
            _row_spec(IED := efeat.shape[1]),
            _full_spec((IVD, VD)), _full_spec((1, VD)),
            _full_spec((IED, ED)), _full_spec((1, ED)),
            _full_spec((VD, QD)), _full_spec((1, QD)),
            _full_spec((ED, QD)), _full_spec((1, QD)),
            _full_spec((ED, VD)), _full_spec((1, VD)),
        ],
        out_specs=[_row_spec(ED), _row_spec(QD), _row_spec(QD), _row_spec(VD)],
        out_shape=[
            jax.ShapeDtypeStruct((N_HEDGES, ED), jnp.float32),
            jax.ShapeDtypeStruct((N_NODES, QD), jnp.float32),
            jax.ShapeDtypeStruct((N_HEDGES, QD), jnp.float32),
            jax.ShapeDtypeStruct((N_HEDGES, VD), jnp.float32),
        ],
    )(vfeat, efeat,
      W_v1.T, b_v1.reshape(1, VD),
      W_e1.T, b_e1.reshape(1, ED),
      W_qv.T, b_qv.reshape(1, QD),
      W_ke.T, b_ke.reshape(1, QD),
      W_ve.T, b_ve.reshape(1, VD))

    kq = jnp.concatenate([k, q], axis=0)
    kq_p = jax.lax.bitcast_convert_type(
        kq.astype(jnp.bfloat16).reshape(2 * N_HEDGES, QD // 2, 2), jnp.int32)
    v3 = v.astype(jnp.bfloat16).reshape(N_HEDGES, VD // 32, 2, _L)
    v_p = jax.lax.bitcast_convert_type(
        jnp.stack([v3[:, :, 0, :], v3[:, :, 1, :]], axis=-1),
        jnp.int32).reshape(N_HEDGES, VD // 2)
    idx_a = jnp.concatenate([src_a, dst_a + N_HEDGES], axis=1)
    attn, wmax = _edge_attn(kq_p, idx_a)
    attn_b = attn.reshape(E_PAD // _CB, _CB)
    acc, den = _edge_agg(v_p, src_b, dst_b, attn_b, wmax)

    h, pred = pl.pallas_call(
        _final_body,
        grid=(grid,),
        in_specs=[
            pl.BlockSpec((_NC, _ROWS, VD), lambda i: (0, i, 0)),
            pl.BlockSpec((_NC, _ROWS, 8), lambda i: (0, i, 0)),
            _full_spec((VD, NCLS)), _full_spec((1, NCLS)),
        ],
        out_specs=[_row_spec(VD), _row_spec(NCLS)],
        out_shape=[
            jax.ShapeDtypeStruct((N_NODES, VD), jnp.float32),
            jax.ShapeDtypeStruct((N_NODES, NCLS), jnp.float32),
        ],
    )(acc, den, W_cls.T, b_cls.reshape(1, NCLS))

    return (h, feat_e, pred)
